# baseline (speedup 1.0000x reference)
import functools

import jax
import jax.numpy as jnp
from jax import lax
from jax.experimental import pallas as pl
from jax.experimental.pallas import tpu as pltpu
from jax.experimental.pallas import tpu_sc as plsc

N = 10000
E = 320000
F = 128
FADJ = 64
FH = 64

NC = 2
NS = 16
NW = NC * NS
L = 16

K = 80
NCHUNK_A = E // NW // K
NCHUNK_B = E // NS // K
NPAD = 10240
RPT = NPAD // NS

_BR = 2048

_SC_MESH = plsc.VectorSubcoreMesh(
    core_axis_name="c", subcore_axis_name="s", num_cores=NC, num_subcores=NS)

_SC_PARAMS = pltpu.CompilerParams(
    needs_layout_passes=False, use_tc_tiling_on_sc=False)


@functools.partial(
    pl.kernel,
    out_type=[
        jax.ShapeDtypeStruct((NW, NCHUNK_A, K), jnp.float32),
        jax.ShapeDtypeStruct((NC, NS, NPAD), jnp.float32),
    ],
    mesh=_SC_MESH,
    compiler_params=_SC_PARAMS,
    scratch_types=[
        pltpu.VMEM((NCHUNK_A, K), jnp.int32),
        pltpu.VMEM((NCHUNK_A, K), jnp.int32),
        pltpu.VMEM((NCHUNK_A, K), jnp.float32),
        [pltpu.VMEM((K, FADJ), jnp.float32) for _ in range(2)],
        [pltpu.VMEM((K, FADJ), jnp.float32) for _ in range(2)],
        pltpu.VMEM((L, L), jnp.float32),
        pltpu.VMEM((NPAD * 4,), jnp.float32),
        pltpu.VMEM((NPAD,), jnp.float32),
        [pltpu.SemaphoreType.DMA for _ in range(2)],
        [pltpu.SemaphoreType.DMA for _ in range(2)],
    ],
)
def _sc_edge_weights(adj_hbm, src_hbm, dst_hbm, ew_hbm, deg_hbm,
                     sidx, didx, ewb, srows2, drows2, dotbuf, degt, degout,
                     sems, semd):
    cid = lax.axis_index("c")
    sid = lax.axis_index("s")
    wid = sid * NC + cid
    lanes = jnp.arange(L, dtype=jnp.int32)

    def zero(r, carry):
        degt[pl.ds(r * L, L)] = jnp.zeros((L,), jnp.float32)
        return carry
    lax.fori_loop(0, NPAD * 4 // L, zero, 0)

    pltpu.sync_copy(src_hbm.at[wid], sidx)
    pltpu.sync_copy(dst_hbm.at[wid], didx)

    def issue(i, b):
        pltpu.async_copy(adj_hbm.at[sidx.at[i]], srows2[b], sems[b])
        pltpu.async_copy(adj_hbm.at[didx.at[i]], drows2[b], semd[b])

    def compute(i, b):
        pltpu.make_async_copy(adj_hbm.at[sidx.at[i]], srows2[b], sems[b]).wait()
        pltpu.make_async_copy(adj_hbm.at[didx.at[i]], drows2[b], semd[b]).wait()
        srows, drows = srows2[b], drows2[b]
        for g in range(K // L):
            for jj in range(L):
                e = g * L + jj
                p = srows[e, pl.ds(0, L)] * drows[e, pl.ds(0, L)]
                for k in range(1, FADJ // L):
                    p = p + srows[e, pl.ds(k * L, L)] * drows[e, pl.ds(k * L, L)]
                dotbuf[jj, pl.ds(0, L)] = plsc.cumsum(p)
            acc = plsc.load_gather(dotbuf, [lanes, jnp.full((L,), L - 1, jnp.int32)])
            ew = 1.0 / (1.0 + jnp.exp(-acc))
            ewb[i, pl.ds(g * L, L)] = ew
            dv = didx[i, pl.ds(g * L, L)]
            idx4 = dv * 4 + (lanes & 3)
            for p_ in range(4):
                m = (lanes >> 2) == p_
                plsc.addupdate_scatter(degt, [idx4], ew, mask=m)

    issue(0, 0)

    def pair(p, carry):
        i0 = 2 * p
        issue(i0 + 1, 1)
        compute(i0, 0)
        issue(i0 + 2, 0)
        compute(i0 + 1, 1)
        return carry

    lax.fori_loop(0, (NCHUNK_A - 1) // 2, pair, 0)
    compute(NCHUNK_A - 1, 0)

    def dred(r, carry):
        nodevec = (lanes + r * L) * 4
        acc = plsc.load_gather(degt, [nodevec])
        for l_ in range(1, 4):
            acc = acc + plsc.load_gather(degt, [nodevec + l_])
        degout[pl.ds(r * L, L)] = acc
        return carry
    lax.fori_loop(0, NPAD // L, dred, 0)

    pltpu.sync_copy(ewb, ew_hbm.at[wid])
    pltpu.sync_copy(degout, deg_hbm.at[cid, sid])


@functools.partial(
    pl.kernel,
    out_type=jax.ShapeDtypeStruct((NC, NPAD, FH), jnp.float32),
    mesh=_SC_MESH,
    compiler_params=_SC_PARAMS,
    scratch_types=[
        pltpu.VMEM((NCHUNK_B, K), jnp.int32),
        pltpu.VMEM((NCHUNK_B, K), jnp.int32),
        pltpu.VMEM((NCHUNK_B, K), jnp.float32),
        [pltpu.VMEM((K, FH), jnp.float32) for _ in range(2)],
        pltpu.VMEM((L, FH), jnp.float32),
        pltpu.VMEM_SHARED((NPAD, FH), jnp.float32),
        [pltpu.SemaphoreType.DMA for _ in range(2)],
    ],
)
def _sc_aggregate(sup_hbm, src_hbm, dst_hbm, ew_hbm, agg_hbm,
                  sidx, didx, ewb, rows2, zb, agg_sh, sems):
    cid = lax.axis_index("c")
    sid = lax.axis_index("s")

    for r in range(L):
        for k in range(FH // L):
            zb[r, pl.ds(k * L, L)] = jnp.zeros((L,), jnp.float32)
    for r in range(RPT // L):
        pltpu.sync_copy(zb, agg_sh.at[pl.ds(sid * RPT + r * L, L)])
    plsc.subcore_barrier()

    pltpu.sync_copy(src_hbm.at[sid], sidx)
    pltpu.sync_copy(dst_hbm.at[sid], didx)
    pltpu.sync_copy(ew_hbm.at[sid], ewb)

    def issue(i, b):
        pltpu.async_copy(sup_hbm.at[cid].at[sidx.at[i]], rows2[b], sems[b])

    def compute(i, b):
        pltpu.make_async_copy(sup_hbm.at[cid].at[sidx.at[i]], rows2[b],
                              sems[b]).wait()
        rows = rows2[b]
        for g in range(K // L):
            ewv = ewb[i, pl.ds(g * L, L)]
            for jj in range(L):
                s = ewv[jj]
                j = g * L + jj
                for k in range(FH // L):
                    rows[j, pl.ds(k * L, L)] = rows[j, pl.ds(k * L, L)] * s
        pltpu.sync_copy(rows, agg_sh.at[didx.at[i]], add=True)

    issue(0, 0)

    def pair(p, carry):
        i0 = 2 * p
        issue(i0 + 1, 1)
        compute(i0, 0)
        issue(i0 + 2, 0)
        compute(i0 + 1, 1)
        return carry

    lax.fori_loop(0, NCHUNK_B // 2 - 1, pair, 0)
    issue(NCHUNK_B - 1, 1)
    compute(NCHUNK_B - 2, 0)
    compute(NCHUNK_B - 1, 1)

    plsc.subcore_barrier()
    pltpu.sync_copy(agg_sh.at[pl.ds(sid * RPT, RPT)],
                    agg_hbm.at[cid, pl.ds(sid * RPT, RPT)])


def _stage1_body(x_ref, wb_ref, bb_ref, wa_ref, ba_ref, w1_ref,
                 feat_ref, adj_ref, sup_ref):
    feat = jnp.maximum(x_ref[...] @ wb_ref[...] + bb_ref[...], 0.0)
    feat_ref[...] = feat
    adj_ref[...] = feat @ wa_ref[...] + ba_ref[...]
    sup = feat @ w1_ref[...]
    sup_ref[0] = sup[:, :FH]
    sup_ref[1] = sup[:, FH:]


def _tc_stage1(x, W_before, b_before, W_adj, b_adj, W1):
    return pl.pallas_call(
        _stage1_body,
        grid=(NPAD // _BR,),
        in_specs=[
            pl.BlockSpec((_BR, F), lambda i: (i, 0)),
            pl.BlockSpec((F, F), lambda i: (0, 0)),
            pl.BlockSpec((1, F), lambda i: (0, 0)),
            pl.BlockSpec((F, FADJ), lambda i: (0, 0)),
            pl.BlockSpec((1, FADJ), lambda i: (0, 0)),
            pl.BlockSpec((F, F), lambda i: (0, 0)),
        ],
        out_specs=[
            pl.BlockSpec((_BR, F), lambda i: (i, 0)),
            pl.BlockSpec((_BR, FADJ), lambda i: (i, 0)),
            pl.BlockSpec((NC, _BR, FH), lambda i: (0, i, 0)),
        ],
        out_shape=[
            jax.ShapeDtypeStruct((NPAD, F), jnp.float32),
            jax.ShapeDtypeStruct((NPAD, FADJ), jnp.float32),
            jax.ShapeDtypeStruct((NC, NPAD, FH), jnp.float32),
        ],
    )(x, W_before, b_before.reshape(1, F), W_adj, b_adj.reshape(1, FADJ), W1)


def _layer1_body(agg_ref, dp_ref, h_ref, b_ref, w_ref,
                 h_out_ref, sup_ref, dinv_ref):
    ones = jnp.ones((NW, 1), jnp.float32)
    deg = jax.lax.dot_general(dp_ref[...], ones, (((0,), (0,)), ((), ())))
    dinv = 1.0 / (deg + 1e-6)
    dinv_ref[...] = dinv
    agg = jnp.concatenate([agg_ref[0], agg_ref[1]], axis=1)
    hn = jnp.tanh(agg * dinv + b_ref[...] + h_ref[...])
    h_out_ref[...] = hn
    sup = hn @ w_ref[...]
    sup_ref[0] = sup[:, :FH]
    sup_ref[1] = sup[:, FH:]


def _tc_layer1(aggp, degp, h, b, W_next):
    return pl.pallas_call(
        _layer1_body,
        grid=(NPAD // _BR,),
        in_specs=[
            pl.BlockSpec((NC, _BR, FH), lambda i: (0, i, 0)),
            pl.BlockSpec((NW, _BR), lambda i: (0, i)),
            pl.BlockSpec((_BR, F), lambda i: (i, 0)),
            pl.BlockSpec((1, F), lambda i: (0, 0)),
            pl.BlockSpec((F, F), lambda i: (0, 0)),
        ],
        out_specs=[
            pl.BlockSpec((_BR, F), lambda i: (i, 0)),
            pl.BlockSpec((NC, _BR, FH), lambda i: (0, i, 0)),
            pl.BlockSpec((_BR, 1), lambda i: (i, 0)),
        ],
        out_shape=[
            jax.ShapeDtypeStruct((NPAD, F), jnp.float32),
            jax.ShapeDtypeStruct((NC, NPAD, FH), jnp.float32),
            jax.ShapeDtypeStruct((NPAD, 1), jnp.float32),
        ],
    )(aggp, degp, h, b.reshape(1, F), W_next)


def _layer2_body(agg_ref, dinv_ref, h_ref, b_ref, w_ref, h_out_ref, sup_ref):
    agg = jnp.concatenate([agg_ref[0], agg_ref[1]], axis=1)
    hn = jnp.tanh(agg * dinv_ref[...] + b_ref[...] + h_ref[...])
    h_out_ref[...] = hn
    sup = hn @ w_ref[...]
    sup_ref[0] = sup[:, :FH]
    sup_ref[1] = sup[:, FH:]


def _tc_layer2(aggp, dinv, h, b, W_next):
    return pl.pallas_call(
        _layer2_body,
        grid=(NPAD // _BR,),
        in_specs=[
            pl.BlockSpec((NC, _BR, FH), lambda i: (0, i, 0)),
            pl.BlockSpec((_BR, 1), lambda i: (i, 0)),
            pl.BlockSpec((_BR, F), lambda i: (i, 0)),
            pl.BlockSpec((1, F), lambda i: (0, 0)),
            pl.BlockSpec((F, F), lambda i: (0, 0)),
        ],
        out_specs=[
            pl.BlockSpec((_BR, F), lambda i: (i, 0)),
            pl.BlockSpec((NC, _BR, FH), lambda i: (0, i, 0)),
        ],
        out_shape=[
            jax.ShapeDtypeStruct((NPAD, F), jnp.float32),
            jax.ShapeDtypeStruct((NC, NPAD, FH), jnp.float32),
        ],
    )(aggp, dinv, h, b.reshape(1, F), W_next)


def _final_body(agg_ref, dinv_ref, h_ref, b_ref, w1_ref, b1_ref,
                w2_ref, b2_ref, out_ref):
    agg = jnp.concatenate([agg_ref[0], agg_ref[1]], axis=1)
    hn = jnp.tanh(agg * dinv_ref[...] + b_ref[...] + h_ref[...])
    t = jnp.maximum(hn @ w1_ref[...] + b1_ref[...], 0.0)
    out_ref[...] = t @ w2_ref[...] + b2_ref[...]


def _tc_final(aggp, dinv, h, b3, W_lin1, b_lin1, W_lin2, b_lin2):
    return pl.pallas_call(
        _final_body,
        grid=(NPAD // _BR,),
        in_specs=[
            pl.BlockSpec((NC, _BR, FH), lambda i: (0, i, 0)),
            pl.BlockSpec((_BR, 1), lambda i: (i, 0)),
            pl.BlockSpec((_BR, F), lambda i: (i, 0)),
            pl.BlockSpec((1, F), lambda i: (0, 0)),
            pl.BlockSpec((F, F), lambda i: (0, 0)),
            pl.BlockSpec((1, F), lambda i: (0, 0)),
            pl.BlockSpec((F, FADJ), lambda i: (0, 0)),
            pl.BlockSpec((1, FADJ), lambda i: (0, 0)),
        ],
        out_specs=pl.BlockSpec((_BR, FADJ), lambda i: (i, 0)),
        out_shape=jax.ShapeDtypeStruct((NPAD, FADJ), jnp.float32),
    )(aggp, dinv, h, b3.reshape(1, F), W_lin1, b_lin1.reshape(1, F),
      W_lin2, b_lin2.reshape(1, FADJ))


def kernel(x, edge_index, W_before, b_before, W_adj, b_adj,
           W1, b1, W2, b2, W3, b3, W_lin1, b_lin1, W_lin2, b_lin2):
    src = edge_index[0]
    dst = edge_index[1]
    src_a = src.reshape(NW, NCHUNK_A, K)
    dst_a = dst.reshape(NW, NCHUNK_A, K)
    src_b = src.reshape(NS, NCHUNK_B, K)
    dst_b = dst.reshape(NS, NCHUNK_B, K)

    x_pad = jnp.pad(x, ((0, NPAD - N), (0, 0)))
    feat, adj_feat, sup1 = _tc_stage1(x_pad, W_before, b_before, W_adj, b_adj, W1)

    ew2d, degp = _sc_edge_weights(adj_feat, src_a, dst_a)
    degp = degp.reshape(NW, NPAD)
    ew_b = ew2d.reshape(NS, NCHUNK_B, K)

    aggp = _sc_aggregate(sup1, src_b, dst_b, ew_b)
    h1, sup2, dinv = _tc_layer1(aggp, degp, feat, b1, W2)

    aggp = _sc_aggregate(sup2, src_b, dst_b, ew_b)
    h2, sup3 = _tc_layer2(aggp, dinv, h1, b2, W3)

    aggp = _sc_aggregate(sup3, src_b, dst_b, ew_b)
    out = _tc_final(aggp, dinv, h2, b3, W_lin1, b_lin1, W_lin2, b_lin2)
    return out[:N]


# speedup vs baseline: 1.0182x; 1.0182x over previous
import functools

import jax
import jax.numpy as jnp
from jax import lax
from jax.experimental import pallas as pl
from jax.experimental.pallas import tpu as pltpu
from jax.experimental.pallas import tpu_sc as plsc

N = 10000
E = 320000
F = 128
FADJ = 64
FH = 64

NC = 2
NS = 16
NW = NC * NS
L = 16

K = 80
NCHUNK_A = E // NW // K
NCHUNK_B = E // NS // K
NPAD = 10240
RPT = NPAD // NS

_BR = 2048

_SC_MESH = plsc.VectorSubcoreMesh(
    core_axis_name="c", subcore_axis_name="s", num_cores=NC, num_subcores=NS)

_SC_PARAMS = pltpu.CompilerParams(
    needs_layout_passes=False, use_tc_tiling_on_sc=False)


@functools.partial(
    pl.kernel,
    out_type=[
        jax.ShapeDtypeStruct((NW, NCHUNK_A, K), jnp.float32),
        jax.ShapeDtypeStruct((NC, NS, NPAD), jnp.float32),
    ],
    mesh=_SC_MESH,
    compiler_params=_SC_PARAMS,
    scratch_types=[
        pltpu.VMEM((NCHUNK_A, K), jnp.int32),
        pltpu.VMEM((NCHUNK_A, K), jnp.int32),
        pltpu.VMEM((NCHUNK_A, K), jnp.float32),
        [pltpu.VMEM((K, FADJ), jnp.float32) for _ in range(2)],
        [pltpu.VMEM((K, FADJ), jnp.float32) for _ in range(2)],
        pltpu.VMEM((L, L), jnp.float32),
        pltpu.VMEM((NPAD * 4,), jnp.float32),
        pltpu.VMEM((NPAD,), jnp.float32),
        [pltpu.SemaphoreType.DMA for _ in range(2)],
        [pltpu.SemaphoreType.DMA for _ in range(2)],
    ],
)
def _sc_edge_weights(adj_hbm, src_hbm, dst_hbm, ew_hbm, deg_hbm,
                     sidx, didx, ewb, srows2, drows2, dotbuf, degt, degout,
                     sems, semd):
    cid = lax.axis_index("c")
    sid = lax.axis_index("s")
    wid = sid * NC + cid
    lanes = jnp.arange(L, dtype=jnp.int32)

    def zero(r, carry):
        degt[pl.ds(r * L, L)] = jnp.zeros((L,), jnp.float32)
        return carry
    lax.fori_loop(0, NPAD * 4 // L, zero, 0)

    pltpu.sync_copy(src_hbm.at[wid], sidx)
    pltpu.sync_copy(dst_hbm.at[wid], didx)

    def issue(i, b):
        pltpu.async_copy(adj_hbm.at[sidx.at[i]], srows2[b], sems[b])
        pltpu.async_copy(adj_hbm.at[didx.at[i]], drows2[b], semd[b])

    def compute(i, b):
        pltpu.make_async_copy(adj_hbm.at[sidx.at[i]], srows2[b], sems[b]).wait()
        pltpu.make_async_copy(adj_hbm.at[didx.at[i]], drows2[b], semd[b]).wait()
        srows, drows = srows2[b], drows2[b]
        for g in range(K // L):
            for jj in range(L):
                e = g * L + jj
                p = srows[e, pl.ds(0, L)] * drows[e, pl.ds(0, L)]
                for k in range(1, FADJ // L):
                    p = p + srows[e, pl.ds(k * L, L)] * drows[e, pl.ds(k * L, L)]
                for sh in (8, 4, 2, 1):
                    p = p + p[lanes ^ sh]
                dotbuf[jj, pl.ds(0, L)] = p
            acc = plsc.load_gather(dotbuf, [lanes, jnp.zeros((L,), jnp.int32)])
            ew = 1.0 / (1.0 + jnp.exp(-acc))
            ewb[i, pl.ds(g * L, L)] = ew
            dv = didx[i, pl.ds(g * L, L)]
            idx4 = dv * 4 + (lanes & 3)
            for p_ in range(4):
                m = (lanes >> 2) == p_
                plsc.addupdate_scatter(degt, [idx4], ew, mask=m)

    issue(0, 0)

    def pair(p, carry):
        i0 = 2 * p
        issue(i0 + 1, 1)
        compute(i0, 0)
        issue(i0 + 2, 0)
        compute(i0 + 1, 1)
        return carry

    lax.fori_loop(0, (NCHUNK_A - 1) // 2, pair, 0)
    compute(NCHUNK_A - 1, 0)

    def dred(r, carry):
        nodevec = (lanes + r * L) * 4
        acc = plsc.load_gather(degt, [nodevec])
        for l_ in range(1, 4):
            acc = acc + plsc.load_gather(degt, [nodevec + l_])
        degout[pl.ds(r * L, L)] = acc
        return carry
    lax.fori_loop(0, NPAD // L, dred, 0)

    pltpu.sync_copy(ewb, ew_hbm.at[wid])
    pltpu.sync_copy(degout, deg_hbm.at[cid, sid])


@functools.partial(
    pl.kernel,
    out_type=jax.ShapeDtypeStruct((NC, NPAD, FH), jnp.float32),
    mesh=_SC_MESH,
    compiler_params=_SC_PARAMS,
    scratch_types=[
        pltpu.VMEM((NCHUNK_B, K), jnp.int32),
        pltpu.VMEM((NCHUNK_B, K), jnp.int32),
        pltpu.VMEM((NCHUNK_B, K), jnp.float32),
        [pltpu.VMEM((K, FH), jnp.float32) for _ in range(2)],
        pltpu.VMEM((L, FH), jnp.float32),
        pltpu.VMEM_SHARED((NPAD, FH), jnp.float32),
        [pltpu.SemaphoreType.DMA for _ in range(2)],
    ],
)
def _sc_aggregate(sup_hbm, src_hbm, dst_hbm, ew_hbm, agg_hbm,
                  sidx, didx, ewb, rows2, zb, agg_sh, sems):
    cid = lax.axis_index("c")
    sid = lax.axis_index("s")

    for r in range(L):
        for k in range(FH // L):
            zb[r, pl.ds(k * L, L)] = jnp.zeros((L,), jnp.float32)
    for r in range(RPT // L):
        pltpu.sync_copy(zb, agg_sh.at[pl.ds(sid * RPT + r * L, L)])
    plsc.subcore_barrier()

    pltpu.sync_copy(src_hbm.at[sid], sidx)
    pltpu.sync_copy(dst_hbm.at[sid], didx)
    pltpu.sync_copy(ew_hbm.at[sid], ewb)

    def issue(i, b):
        pltpu.async_copy(sup_hbm.at[cid].at[sidx.at[i]], rows2[b], sems[b])

    def compute(i, b):
        pltpu.make_async_copy(sup_hbm.at[cid].at[sidx.at[i]], rows2[b],
                              sems[b]).wait()
        rows = rows2[b]
        for g in range(K // L):
            ewv = ewb[i, pl.ds(g * L, L)]
            for jj in range(L):
                s = ewv[jj]
                j = g * L + jj
                for k in range(FH // L):
                    rows[j, pl.ds(k * L, L)] = rows[j, pl.ds(k * L, L)] * s
        pltpu.sync_copy(rows, agg_sh.at[didx.at[i]], add=True)

    issue(0, 0)

    def pair(p, carry):
        i0 = 2 * p
        issue(i0 + 1, 1)
        compute(i0, 0)
        issue(i0 + 2, 0)
        compute(i0 + 1, 1)
        return carry

    lax.fori_loop(0, NCHUNK_B // 2 - 1, pair, 0)
    issue(NCHUNK_B - 1, 1)
    compute(NCHUNK_B - 2, 0)
    compute(NCHUNK_B - 1, 1)

    plsc.subcore_barrier()
    pltpu.sync_copy(agg_sh.at[pl.ds(sid * RPT, RPT)],
                    agg_hbm.at[cid, pl.ds(sid * RPT, RPT)])


def _stage1_body(x_ref, wb_ref, bb_ref, wa_ref, ba_ref, w1_ref,
                 feat_ref, adj_ref, sup_ref):
    feat = jnp.maximum(x_ref[...] @ wb_ref[...] + bb_ref[...], 0.0)
    feat_ref[...] = feat
    adj_ref[...] = feat @ wa_ref[...] + ba_ref[...]
    sup = feat @ w1_ref[...]
    sup_ref[0] = sup[:, :FH]
    sup_ref[1] = sup[:, FH:]


def _tc_stage1(x, W_before, b_before, W_adj, b_adj, W1):
    return pl.pallas_call(
        _stage1_body,
        grid=(NPAD // _BR,),
        in_specs=[
            pl.BlockSpec((_BR, F), lambda i: (i, 0)),
            pl.BlockSpec((F, F), lambda i: (0, 0)),
            pl.BlockSpec((1, F), lambda i: (0, 0)),
            pl.BlockSpec((F, FADJ), lambda i: (0, 0)),
            pl.BlockSpec((1, FADJ), lambda i: (0, 0)),
            pl.BlockSpec((F, F), lambda i: (0, 0)),
        ],
        out_specs=[
            pl.BlockSpec((_BR, F), lambda i: (i, 0)),
            pl.BlockSpec((_BR, FADJ), lambda i: (i, 0)),
            pl.BlockSpec((NC, _BR, FH), lambda i: (0, i, 0)),
        ],
        out_shape=[
            jax.ShapeDtypeStruct((NPAD, F), jnp.float32),
            jax.ShapeDtypeStruct((NPAD, FADJ), jnp.float32),
            jax.ShapeDtypeStruct((NC, NPAD, FH), jnp.float32),
        ],
    )(x, W_before, b_before.reshape(1, F), W_adj, b_adj.reshape(1, FADJ), W1)


def _layer1_body(agg_ref, dp_ref, h_ref, b_ref, w_ref,
                 h_out_ref, sup_ref, dinv_ref):
    ones = jnp.ones((NW, 1), jnp.float32)
    deg = jax.lax.dot_general(dp_ref[...], ones, (((0,), (0,)), ((), ())))
    dinv = 1.0 / (deg + 1e-6)
    dinv_ref[...] = dinv
    agg = jnp.concatenate([agg_ref[0], agg_ref[1]], axis=1)
    hn = jnp.tanh(agg * dinv + b_ref[...] + h_ref[...])
    h_out_ref[...] = hn
    sup = hn @ w_ref[...]
    sup_ref[0] = sup[:, :FH]
    sup_ref[1] = sup[:, FH:]


def _tc_layer1(aggp, degp, h, b, W_next):
    return pl.pallas_call(
        _layer1_body,
        grid=(NPAD // _BR,),
        in_specs=[
            pl.BlockSpec((NC, _BR, FH), lambda i: (0, i, 0)),
            pl.BlockSpec((NW, _BR), lambda i: (0, i)),
            pl.BlockSpec((_BR, F), lambda i: (i, 0)),
            pl.BlockSpec((1, F), lambda i: (0, 0)),
            pl.BlockSpec((F, F), lambda i: (0, 0)),
        ],
        out_specs=[
            pl.BlockSpec((_BR, F), lambda i: (i, 0)),
            pl.BlockSpec((NC, _BR, FH), lambda i: (0, i, 0)),
            pl.BlockSpec((_BR, 1), lambda i: (i, 0)),
        ],
        out_shape=[
            jax.ShapeDtypeStruct((NPAD, F), jnp.float32),
            jax.ShapeDtypeStruct((NC, NPAD, FH), jnp.float32),
            jax.ShapeDtypeStruct((NPAD, 1), jnp.float32),
        ],
    )(aggp, degp, h, b.reshape(1, F), W_next)


def _layer2_body(agg_ref, dinv_ref, h_ref, b_ref, w_ref, h_out_ref, sup_ref):
    agg = jnp.concatenate([agg_ref[0], agg_ref[1]], axis=1)
    hn = jnp.tanh(agg * dinv_ref[...] + b_ref[...] + h_ref[...])
    h_out_ref[...] = hn
    sup = hn @ w_ref[...]
    sup_ref[0] = sup[:, :FH]
    sup_ref[1] = sup[:, FH:]


def _tc_layer2(aggp, dinv, h, b, W_next):
    return pl.pallas_call(
        _layer2_body,
        grid=(NPAD // _BR,),
        in_specs=[
            pl.BlockSpec((NC, _BR, FH), lambda i: (0, i, 0)),
            pl.BlockSpec((_BR, 1), lambda i: (i, 0)),
            pl.BlockSpec((_BR, F), lambda i: (i, 0)),
            pl.BlockSpec((1, F), lambda i: (0, 0)),
            pl.BlockSpec((F, F), lambda i: (0, 0)),
        ],
        out_specs=[
            pl.BlockSpec((_BR, F), lambda i: (i, 0)),
            pl.BlockSpec((NC, _BR, FH), lambda i: (0, i, 0)),
        ],
        out_shape=[
            jax.ShapeDtypeStruct((NPAD, F), jnp.float32),
            jax.ShapeDtypeStruct((NC, NPAD, FH), jnp.float32),
        ],
    )(aggp, dinv, h, b.reshape(1, F), W_next)


def _final_body(agg_ref, dinv_ref, h_ref, b_ref, w1_ref, b1_ref,
                w2_ref, b2_ref, out_ref):
    agg = jnp.concatenate([agg_ref[0], agg_ref[1]], axis=1)
    hn = jnp.tanh(agg * dinv_ref[...] + b_ref[...] + h_ref[...])
    t = jnp.maximum(hn @ w1_ref[...] + b1_ref[...], 0.0)
    out_ref[...] = t @ w2_ref[...] + b2_ref[...]


def _tc_final(aggp, dinv, h, b3, W_lin1, b_lin1, W_lin2, b_lin2):
    return pl.pallas_call(
        _final_body,
        grid=(NPAD // _BR,),
        in_specs=[
            pl.BlockSpec((NC, _BR, FH), lambda i: (0, i, 0)),
            pl.BlockSpec((_BR, 1), lambda i: (i, 0)),
            pl.BlockSpec((_BR, F), lambda i: (i, 0)),
            pl.BlockSpec((1, F), lambda i: (0, 0)),
            pl.BlockSpec((F, F), lambda i: (0, 0)),
            pl.BlockSpec((1, F), lambda i: (0, 0)),
            pl.BlockSpec((F, FADJ), lambda i: (0, 0)),
            pl.BlockSpec((1, FADJ), lambda i: (0, 0)),
        ],
        out_specs=pl.BlockSpec((_BR, FADJ), lambda i: (i, 0)),
        out_shape=jax.ShapeDtypeStruct((NPAD, FADJ), jnp.float32),
    )(aggp, dinv, h, b3.reshape(1, F), W_lin1, b_lin1.reshape(1, F),
      W_lin2, b_lin2.reshape(1, FADJ))


def kernel(x, edge_index, W_before, b_before, W_adj, b_adj,
           W1, b1, W2, b2, W3, b3, W_lin1, b_lin1, W_lin2, b_lin2):
    src = edge_index[0]
    dst = edge_index[1]
    src_a = src.reshape(NW, NCHUNK_A, K)
    dst_a = dst.reshape(NW, NCHUNK_A, K)
    src_b = src.reshape(NS, NCHUNK_B, K)
    dst_b = dst.reshape(NS, NCHUNK_B, K)

    x_pad = jnp.pad(x, ((0, NPAD - N), (0, 0)))
    feat, adj_feat, sup1 = _tc_stage1(x_pad, W_before, b_before, W_adj, b_adj, W1)

    ew2d, degp = _sc_edge_weights(adj_feat, src_a, dst_a)
    degp = degp.reshape(NW, NPAD)
    ew_b = ew2d.reshape(NS, NCHUNK_B, K)

    aggp = _sc_aggregate(sup1, src_b, dst_b, ew_b)
    h1, sup2, dinv = _tc_layer1(aggp, degp, feat, b1, W2)

    aggp = _sc_aggregate(sup2, src_b, dst_b, ew_b)
    h2, sup3 = _tc_layer2(aggp, dinv, h1, b2, W3)

    aggp = _sc_aggregate(sup3, src_b, dst_b, ew_b)
    out = _tc_final(aggp, dinv, h2, b3, W_lin1, b_lin1, W_lin2, b_lin2)
    return out[:N]
